# Initial kernel scaffold; baseline (speedup 1.0000x reference)
import jax
import jax.numpy as jnp
from jax.experimental import pallas as pl


def kernel(feat, edge_index, norm_data, weight_n, weight_s):
    raise NotImplementedError("write your pallas kernel here")


# speedup vs baseline: 7.5600x; 7.5600x over previous
import jax
import jax.numpy as jnp
from jax import lax
from jax.experimental import pallas as pl
from jax.experimental.pallas import tpu as pltpu
from jax.experimental.pallas import tpu_sc as plsc

N = 10000
E = 320000
D = 128

NC = 2
NS = 16
NW = NC * NS
EPW = E // NW
CHUNK = 80
KCH = EPW // CHUNK
NPAD = 10240
RPT = NPAD // NS


def _sc_agg_body(feat_hbm, src_hbm, dst_hbm, zeros_hbm, out_hbm,
                 sidx, didx, rows, sem, acc):
    c = lax.axis_index("c")
    s = lax.axis_index("s")
    wid = s * NC + c

    pltpu.sync_copy(src_hbm.at[wid], sidx)
    pltpu.sync_copy(dst_hbm.at[wid], didx)
    pltpu.sync_copy(zeros_hbm, acc.at[pl.ds(s * RPT, RPT)])
    plsc.subcore_barrier()

    def body(j, _):
        pltpu.async_copy(feat_hbm.at[sidx.at[j]], rows, sem).wait()
        pltpu.sync_copy(rows, acc.at[didx.at[j]], add=True)
        return 0

    lax.fori_loop(0, KCH, body, 0)
    plsc.subcore_barrier()

    pltpu.sync_copy(acc.at[pl.ds(s * RPT, RPT)],
                    out_hbm.at[c, pl.ds(s * RPT, RPT)])


@jax.jit
def _sc_agg(feat, src3, dst3, zeros):
    mesh = plsc.VectorSubcoreMesh(core_axis_name="c", subcore_axis_name="s",
                                  num_cores=NC, num_subcores=NS)
    f = pl.kernel(
        _sc_agg_body,
        out_type=jax.ShapeDtypeStruct((NC, NPAD, D), jnp.float32),
        mesh=mesh,
        scratch_types=[
            pltpu.VMEM((KCH, CHUNK), jnp.int32),
            pltpu.VMEM((KCH, CHUNK), jnp.int32),
            pltpu.VMEM((CHUNK, D), jnp.float32),
            pltpu.SemaphoreType.DMA,
            pltpu.VMEM_SHARED((NPAD, D), jnp.float32),
        ],
    )
    return f(feat, src3, dst3, zeros)


def _tc_combine_body(p_ref, feat_ref, norm_ref, wn_ref, ws_ref, out_ref):
    ssum = p_ref[0] + p_ref[1]
    agg = lax.dot(ssum, wn_ref[...], preferred_element_type=jnp.float32)
    self_part = lax.dot(feat_ref[...], ws_ref[...],
                        preferred_element_type=jnp.float32)
    out_ref[...] = agg * norm_ref[...] + self_part


@jax.jit
def _tc_combine(partials, feat, norm, wn, ws):
    blk = 1000
    grid = (N // blk,)
    return pl.pallas_call(
        _tc_combine_body,
        grid=grid,
        in_specs=[
            pl.BlockSpec((NC, blk, D), lambda i: (0, i, 0)),
            pl.BlockSpec((blk, D), lambda i: (i, 0)),
            pl.BlockSpec((blk, 1), lambda i: (i, 0)),
            pl.BlockSpec((D, D), lambda i: (0, 0)),
            pl.BlockSpec((D, D), lambda i: (0, 0)),
        ],
        out_specs=pl.BlockSpec((blk, D), lambda i: (i, 0)),
        out_shape=jax.ShapeDtypeStruct((N, D), jnp.float32),
    )(partials, feat, norm, wn, ws)


def kernel(feat, edge_index, norm_data, weight_n, weight_s):
    src3 = edge_index[0].astype(jnp.int32).reshape(NW, KCH, CHUNK)
    dst3 = edge_index[1].astype(jnp.int32).reshape(NW, KCH, CHUNK)
    zeros = jnp.zeros((RPT, D), jnp.float32)
    partials = _sc_agg(feat, src3, dst3, zeros)
    return _tc_combine(partials, feat, norm_data, weight_n, weight_s)


# speedup vs baseline: 9.7192x; 1.2856x over previous
import jax
import jax.numpy as jnp
from jax import lax
from jax.experimental import pallas as pl
from jax.experimental.pallas import tpu as pltpu
from jax.experimental.pallas import tpu_sc as plsc

N = 10000
E = 320000
D = 128

NC = 2
NS = 16
NW = NC * NS
EPW = E // NW
CHUNK = 80
KCH = EPW // CHUNK
NPAD = 10240
RPT = NPAD // NS


def _sc_agg_body(feat_hbm, src_hbm, dst_hbm, zeros_hbm, out_hbm,
                 s0, s1, didx, rows0, rows1, sem0, sem1, acc):
    c = lax.axis_index("c")
    s = lax.axis_index("s")
    wid = s * NC + c

    pltpu.sync_copy(dst_hbm.at[wid], didx)
    pltpu.sync_copy(zeros_hbm, acc.at[pl.ds(s * RPT, RPT)])
    plsc.subcore_barrier()

    pltpu.sync_copy(src_hbm.at[wid, 0], s0)
    pltpu.async_copy(feat_hbm.at[s0.at[0]], rows0, sem0)
    pltpu.sync_copy(src_hbm.at[wid, 1], s1)
    pltpu.async_copy(feat_hbm.at[s1.at[0]], rows1, sem1)

    def body(g, _):
        j0 = 2 * g
        j1 = j0 + 1
        pltpu.make_async_copy(feat_hbm.at[s0.at[0]], rows0, sem0).wait()
        pltpu.sync_copy(rows0, acc.at[didx.at[j0]], add=True)

        @pl.when(j0 + 2 < KCH)
        def _():
            pltpu.sync_copy(src_hbm.at[wid, j0 + 2], s0)
            pltpu.async_copy(feat_hbm.at[s0.at[0]], rows0, sem0)

        pltpu.make_async_copy(feat_hbm.at[s1.at[0]], rows1, sem1).wait()
        pltpu.sync_copy(rows1, acc.at[didx.at[j1]], add=True)

        @pl.when(j1 + 2 < KCH)
        def _():
            pltpu.sync_copy(src_hbm.at[wid, j1 + 2], s1)
            pltpu.async_copy(feat_hbm.at[s1.at[0]], rows1, sem1)

        return 0

    lax.fori_loop(0, KCH // 2, body, 0)
    last = KCH - 1
    pltpu.make_async_copy(feat_hbm.at[s0.at[0]], rows0, sem0).wait()
    pltpu.sync_copy(rows0, acc.at[didx.at[last]], add=True)
    plsc.subcore_barrier()

    pltpu.sync_copy(acc.at[pl.ds(s * RPT, RPT)],
                    out_hbm.at[c, pl.ds(s * RPT, RPT)])


@jax.jit
def _sc_agg(feat, src3, dst3, zeros):
    mesh = plsc.VectorSubcoreMesh(core_axis_name="c", subcore_axis_name="s",
                                  num_cores=NC, num_subcores=NS)
    f = pl.kernel(
        _sc_agg_body,
        out_type=jax.ShapeDtypeStruct((NC, NPAD, D), jnp.float32),
        mesh=mesh,
        scratch_types=[
            pltpu.VMEM((1, CHUNK), jnp.int32),
            pltpu.VMEM((1, CHUNK), jnp.int32),
            pltpu.VMEM((KCH, CHUNK), jnp.int32),
            pltpu.VMEM((CHUNK, D), jnp.float32),
            pltpu.VMEM((CHUNK, D), jnp.float32),
            pltpu.SemaphoreType.DMA,
            pltpu.SemaphoreType.DMA,
            pltpu.VMEM_SHARED((NPAD, D), jnp.float32),
        ],
    )
    return f(feat, src3, dst3, zeros)


def _tc_combine_body(p_ref, feat_ref, norm_ref, wn_ref, ws_ref, out_ref):
    ssum = p_ref[0] + p_ref[1]
    agg = lax.dot(ssum, wn_ref[...], preferred_element_type=jnp.float32)
    self_part = lax.dot(feat_ref[...], ws_ref[...],
                        preferred_element_type=jnp.float32)
    out_ref[...] = agg * norm_ref[...] + self_part


@jax.jit
def _tc_combine(partials, feat, norm, wn, ws):
    blk = 1000
    grid = (N // blk,)
    return pl.pallas_call(
        _tc_combine_body,
        grid=grid,
        in_specs=[
            pl.BlockSpec((NC, blk, D), lambda i: (0, i, 0)),
            pl.BlockSpec((blk, D), lambda i: (i, 0)),
            pl.BlockSpec((blk, 1), lambda i: (i, 0)),
            pl.BlockSpec((D, D), lambda i: (0, 0)),
            pl.BlockSpec((D, D), lambda i: (0, 0)),
        ],
        out_specs=pl.BlockSpec((blk, D), lambda i: (i, 0)),
        out_shape=jax.ShapeDtypeStruct((N, D), jnp.float32),
    )(partials, feat, norm, wn, ws)


def kernel(feat, edge_index, norm_data, weight_n, weight_s):
    src3 = edge_index[0].astype(jnp.int32).reshape(NW, KCH, 1, CHUNK)
    dst3 = edge_index[1].astype(jnp.int32).reshape(NW, KCH, CHUNK)
    zeros = jnp.zeros((RPT, D), jnp.float32)
    partials = _sc_agg(feat, src3, dst3, zeros)
    return _tc_combine(partials, feat, norm_data, weight_n, weight_s)
